# baseline (speedup 1.0000x reference)
import functools

import jax
import jax.numpy as jnp
from jax import lax
from jax.experimental import pallas as pl
from jax.experimental.pallas import tpu as pltpu
from jax.experimental.pallas import tpu_sc as plsc

N = 10000
D = 256
KF = 4
DD = 64
M = 160000
ROUTIT = 6
NT = 32
NR = 313
N_PAD = NT * NR
DUMMY = 10008
E = 80
GROUPS = E // 16
M_PAD = M + NT * E
NOFF = 48


def _tc_norm_body(x_ref, o_ref):
    x = x_ref[...]
    for f in range(KF):
        xs = x[:, f * DD:(f + 1) * DD]
        s = jnp.sum(xs * xs, axis=1, keepdims=True)
        nrm = jnp.maximum(jnp.sqrt(s), 1e-12)
        o_ref[:, f * DD:(f + 1) * DD] = xs / nrm


def _tc_normalize(c):
    return pl.pallas_call(
        _tc_norm_body,
        grid=(4,),
        in_specs=[pl.BlockSpec((N_PAD // 4, D), lambda i: (i, 0))],
        out_specs=pl.BlockSpec((N_PAD // 4, D), lambda i: (i, 0)),
        out_shape=jax.ShapeDtypeStruct((N_PAD, D), jnp.float32),
    )(c)


_MESH = plsc.VectorSubcoreMesh(core_axis_name="c", subcore_axis_name="s")


@functools.partial(
    pl.kernel,
    mesh=_MESH,
    compiler_params=pltpu.CompilerParams(use_tc_tiling_on_sc=False,
                                         needs_layout_passes=False),
    out_type=jax.ShapeDtypeStruct((N_PAD, D), jnp.float32),
    scratch_types=[
        pltpu.VMEM((NR + 15, D), jnp.float32),
        pltpu.VMEM((NOFF,), jnp.int32),
        pltpu.VMEM((E,), jnp.int32),
        pltpu.VMEM((E,), jnp.int32),
        pltpu.VMEM((E, D), jnp.float32),
        pltpu.VMEM((E, D), jnp.float32),
        pltpu.SemaphoreType.DMA,
        pltpu.SemaphoreType.DMA,
    ],
)
def _sc_route(c0, ccur, src, trg, off_h, out, acc, off_v, src_v, trg_v,
              zbuf, cbuf, sem1, sem2):
    cid = lax.axis_index("c")
    sid = lax.axis_index("s")
    wid = cid * 16 + sid
    nbase = wid * NR
    lanes = lax.iota(jnp.int32, 16)

    pltpu.sync_copy(ccur.at[pl.ds(nbase, NR)], acc.at[pl.ds(0, NR)])
    pltpu.sync_copy(off_h, off_v)

    def vscalar(idx):
        win = off_v[pl.ds((idx // 16) * 16, 16)]
        sel = jnp.where(lanes == (idx % 16), win, 0)
        return jnp.sum(sel)

    o0 = vscalar(wid)
    o1 = vscalar(wid + 1)
    nchunks = (o1 - o0) // E

    def chunk_body(i, carry):
        cb = pl.multiple_of(o0 + i * E, 8)
        pltpu.sync_copy(src.at[pl.ds(cb, E)], src_v)
        pltpu.sync_copy(trg.at[pl.ds(cb, E)], trg_v)
        pltpu.async_copy(c0.at[src_v], zbuf, sem1).wait()
        pltpu.async_copy(ccur.at[trg_v], cbuf, sem2).wait()

        def group_body(g, gc):
            rows = g * 16 + lanes
            tv = trg_v[pl.ds(g * 16, 16)]
            soff = jnp.minimum(jnp.maximum(tv - nbase, 0), NR)

            ps = []
            for f in range(KF):
                def dot_block(b, pacc, f=f):
                    colv = jnp.full((16,), f * DD, jnp.int32) + b * 16
                    acc16 = pacc
                    for j in range(16):
                        col = colv + ((lanes + j) & 15)
                        zc = plsc.load_gather(zbuf, [rows, col])
                        cc = plsc.load_gather(cbuf, [rows, col])
                        acc16 = acc16 + zc * cc
                    return acc16
                ps.append(lax.fori_loop(0, DD // 16, dot_block,
                                        jnp.zeros((16,), jnp.float32)))

            mx = jnp.maximum(jnp.maximum(ps[0], ps[1]),
                             jnp.maximum(ps[2], ps[3]))
            es = [jnp.exp(p - mx) for p in ps]
            ssum = es[0] + es[1] + es[2] + es[3]
            rinv = 1.0 / ssum
            ws = [e * rinv for e in es]

            for f in range(KF):
                def w_block(b, carry2, f=f):
                    colv = jnp.full((16,), f * DD, jnp.int32) + b * 16
                    for j in range(16):
                        col = colv + ((lanes + j) & 15)
                        zc = plsc.load_gather(zbuf, [rows, col])
                        plsc.addupdate_scatter(acc, [soff, col],
                                               zc * ws[f])
                    return carry2
                lax.fori_loop(0, DD // 16, w_block, 0)
            return gc

        lax.fori_loop(0, GROUPS, group_body, 0)
        return carry

    lax.fori_loop(0, nchunks, chunk_body, 0)

    pltpu.sync_copy(acc.at[pl.ds(0, NR)], out.at[pl.ds(nbase, NR)])


def _prepare_edges(trg, src):
    bucket = trg // NR
    B = 128
    nb = M // B
    brs = bucket.reshape(nb, B)
    oh = (brs[:, :, None] == jnp.arange(NT, dtype=jnp.int32)).astype(jnp.int32)
    intra = jnp.cumsum(oh, axis=1)
    blk = intra[:, -1, :]
    blk_off = jnp.concatenate(
        [jnp.zeros((1, NT), jnp.int32),
         jnp.cumsum(blk[:-1], axis=0).astype(jnp.int32)])
    counts = blk_off[-1] + blk[-1]
    padded = ((counts + E - 1) // E) * E
    off = jnp.concatenate(
        [jnp.zeros((1,), jnp.int32),
         jnp.cumsum(padded).astype(jnp.int32)])
    rank = (jnp.take_along_axis(intra, brs[:, :, None], axis=2)[:, :, 0]
            + jnp.take_along_axis(blk_off, brs, axis=1) - 1)
    pos = (off[bucket] + rank.reshape(M)).astype(jnp.int32)
    trg_p = jnp.full((M_PAD,), DUMMY, jnp.int32).at[pos].set(
        trg, unique_indices=True)
    src_p = jnp.full((M_PAD,), DUMMY, jnp.int32).at[pos].set(
        src, unique_indices=True)
    off48 = jnp.pad(off, (0, NOFF - NT - 1), mode="edge")
    return trg_p, src_p, off48.astype(jnp.int32)


def kernel(x, src_trg):
    trg = src_trg[0]
    src = src_trg[1]
    trg_p, src_p, off48 = _prepare_edges(trg, src)
    x_pad = jnp.pad(x, ((0, N_PAD - N), (0, 0)))
    c0 = _tc_normalize(x_pad)
    c = c0
    for _ in range(ROUTIT):
        acc = _sc_route(c0, c, src_p, trg_p, off48)
        c = _tc_normalize(acc)
    return c[:N]


# speedup vs baseline: 1.3515x; 1.3515x over previous
import functools

import jax
import jax.numpy as jnp
from jax import lax
from jax.experimental import pallas as pl
from jax.experimental.pallas import tpu as pltpu
from jax.experimental.pallas import tpu_sc as plsc

N = 10000
D = 256
KF = 4
DD = 64
M = 160000
ROUTIT = 6
NT = 32
NR = 313
N_PAD = NT * NR
DUMMY = 10008
E = 80
GROUPS = E // 16
M_PAD = M + NT * E
NOFF = 48


def _tc_norm_body(x_ref, o_ref):
    x = x_ref[...]
    for f in range(KF):
        xs = x[:, f * DD:(f + 1) * DD]
        s = jnp.sum(xs * xs, axis=1, keepdims=True)
        nrm = jnp.maximum(jnp.sqrt(s), 1e-12)
        o_ref[:, f * DD:(f + 1) * DD] = xs / nrm


def _tc_normalize(c):
    return pl.pallas_call(
        _tc_norm_body,
        grid=(4,),
        in_specs=[pl.BlockSpec((N_PAD // 4, D), lambda i: (i, 0))],
        out_specs=pl.BlockSpec((N_PAD // 4, D), lambda i: (i, 0)),
        out_shape=jax.ShapeDtypeStruct((N_PAD, D), jnp.float32),
    )(c)


_MESH = plsc.VectorSubcoreMesh(core_axis_name="c", subcore_axis_name="s")


@functools.partial(
    pl.kernel,
    mesh=_MESH,
    compiler_params=pltpu.CompilerParams(use_tc_tiling_on_sc=False,
                                         needs_layout_passes=False),
    out_type=jax.ShapeDtypeStruct((N_PAD, D), jnp.float32),
    scratch_types=[
        pltpu.VMEM((NR + 15, D), jnp.float32),
        pltpu.VMEM((NOFF,), jnp.int32),
        pltpu.VMEM((E,), jnp.int32),
        pltpu.VMEM((E,), jnp.int32),
        pltpu.VMEM((E, D), jnp.float32),
        pltpu.VMEM((E, D), jnp.float32),
        pltpu.SemaphoreType.DMA,
        pltpu.SemaphoreType.DMA,
    ],
)
def _sc_route(c0, ccur, src, trg, off_h, out, acc, off_v, src_v, trg_v,
              zbuf, cbuf, sem1, sem2):
    cid = lax.axis_index("c")
    sid = lax.axis_index("s")
    wid = cid * 16 + sid
    nbase = wid * NR
    lanes = lax.iota(jnp.int32, 16)

    pltpu.sync_copy(ccur.at[pl.ds(nbase, NR)], acc.at[pl.ds(0, NR)])
    pltpu.sync_copy(off_h, off_v)

    def vscalar(idx):
        win = off_v[pl.ds((idx // 16) * 16, 16)]
        sel = jnp.where(lanes == (idx % 16), win, 0)
        return jnp.sum(sel)

    o0 = vscalar(wid)
    o1 = vscalar(wid + 1)
    nchunks = (o1 - o0) // E

    def chunk_body(i, carry):
        cb = pl.multiple_of(o0 + i * E, 8)
        pltpu.sync_copy(src.at[pl.ds(cb, E)], src_v)
        pltpu.sync_copy(trg.at[pl.ds(cb, E)], trg_v)
        pltpu.async_copy(c0.at[src_v], zbuf, sem1).wait()
        pltpu.async_copy(ccur.at[trg_v], cbuf, sem2).wait()

        def group_body(g, gc):
            rows = g * 16 + lanes
            tv = trg_v[pl.ds(g * 16, 16)]
            soff = jnp.minimum(jnp.maximum(tv - nbase, 0), NR)

            ps = []
            for f in range(KF):
                def dot_block(b, pacc, f=f):
                    colv = jnp.full((16,), f * DD, jnp.int32) + b * 16
                    acc16 = pacc
                    for j in range(16):
                        col = colv + ((lanes + j) & 15)
                        zc = plsc.load_gather(zbuf, [rows, col])
                        cc = plsc.load_gather(cbuf, [rows, col])
                        acc16 = acc16 + zc * cc
                    return acc16
                ps.append(lax.fori_loop(0, DD // 16, dot_block,
                                        jnp.zeros((16,), jnp.float32)))

            mx = jnp.maximum(jnp.maximum(ps[0], ps[1]),
                             jnp.maximum(ps[2], ps[3]))
            es = [jnp.exp(p - mx) for p in ps]
            ssum = es[0] + es[1] + es[2] + es[3]
            rinv = 1.0 / ssum
            ws = [e * rinv for e in es]

            for f in range(KF):
                def w_block(b, carry2, f=f):
                    colv = jnp.full((16,), f * DD, jnp.int32) + b * 16
                    for j in range(16):
                        col = colv + ((lanes + j) & 15)
                        zc = plsc.load_gather(zbuf, [rows, col])
                        plsc.addupdate_scatter(acc, [soff, col],
                                               zc * ws[f])
                    return carry2
                lax.fori_loop(0, DD // 16, w_block, 0)
            return gc

        lax.fori_loop(0, GROUPS, group_body, 0)
        return carry

    lax.fori_loop(0, nchunks, chunk_body, 0)

    pltpu.sync_copy(acc.at[pl.ds(0, NR)], out.at[pl.ds(nbase, NR)])


def _prepare_edges(trg, src):
    bucket = trg // NR
    key = bucket * 262144 + jnp.arange(M, dtype=jnp.int32)
    ks = jnp.sort(key)
    order = ks & 262143
    b_s = ks >> 18
    start = jnp.searchsorted(b_s, jnp.arange(NT + 1)).astype(jnp.int32)
    counts = start[1:] - start[:-1]
    padded = ((counts + E - 1) // E) * E
    off = jnp.concatenate(
        [jnp.zeros((1,), jnp.int32),
         jnp.cumsum(padded).astype(jnp.int32)])
    j = jnp.arange(M_PAD, dtype=jnp.int32)
    bj = (jnp.searchsorted(off, j, side="right") - 1).astype(jnp.int32)
    gidx = start[bj] + (j - off[bj])
    real = gidx < start[bj + 1]
    gclamp = jnp.minimum(gidx, M - 1)
    trg_sorted = trg[order]
    src_sorted = src[order]
    trg_p = jnp.where(real, trg_sorted[gclamp], DUMMY)
    src_p = jnp.where(real, src_sorted[gclamp], DUMMY)
    off48 = jnp.pad(off, (0, NOFF - NT - 1), mode="edge")
    return trg_p, src_p, off48.astype(jnp.int32)


def kernel(x, src_trg):
    trg = src_trg[0]
    src = src_trg[1]
    trg_p, src_p, off48 = _prepare_edges(trg, src)
    x_pad = jnp.pad(x, ((0, N_PAD - N), (0, 0)))
    c0 = _tc_normalize(x_pad)
    c = c0
    for _ in range(ROUTIT):
        acc = _sc_route(c0, c, src_p, trg_p, off48)
        c = _tc_normalize(acc)
    return c[:N]


# speedup vs baseline: 1.4234x; 1.0532x over previous
import functools

import jax
import jax.numpy as jnp
from jax import lax
from jax.experimental import pallas as pl
from jax.experimental.pallas import tpu as pltpu
from jax.experimental.pallas import tpu_sc as plsc

N = 10000
D = 256
KF = 4
DD = 64
M = 160000
ROUTIT = 6
NT = 32
NR = 313
N_PAD = NT * NR
DUMMY = 10008
E = 80
GROUPS = E // 16
M_PAD = M + NT * E
NOFF = 48


def _tc_norm_body(x_ref, o_ref):
    x = x_ref[...]
    for f in range(KF):
        xs = x[:, f * DD:(f + 1) * DD]
        s = jnp.sum(xs * xs, axis=1, keepdims=True)
        nrm = jnp.maximum(jnp.sqrt(s), 1e-12)
        o_ref[:, f * DD:(f + 1) * DD] = xs / nrm


def _tc_normalize(c):
    return pl.pallas_call(
        _tc_norm_body,
        grid=(4,),
        in_specs=[pl.BlockSpec((N_PAD // 4, D), lambda i: (i, 0))],
        out_specs=pl.BlockSpec((N_PAD // 4, D), lambda i: (i, 0)),
        out_shape=jax.ShapeDtypeStruct((N_PAD, D), jnp.float32),
    )(c)


_MESH = plsc.VectorSubcoreMesh(core_axis_name="c", subcore_axis_name="s")


@functools.partial(
    pl.kernel,
    mesh=_MESH,
    compiler_params=pltpu.CompilerParams(use_tc_tiling_on_sc=False,
                                         needs_layout_passes=False),
    out_type=jax.ShapeDtypeStruct((N_PAD, D), jnp.float32),
    scratch_types=[
        pltpu.VMEM((NR + 15, D), jnp.float32),
        pltpu.VMEM((NOFF,), jnp.int32),
        pltpu.VMEM((E,), jnp.int32),
        pltpu.VMEM((E,), jnp.int32),
        pltpu.VMEM((E, D), jnp.float32),
        pltpu.VMEM((E, D), jnp.float32),
        pltpu.SemaphoreType.DMA,
        pltpu.SemaphoreType.DMA,
    ],
)
def _sc_route(c0, ccur, src, trg, off_h, out, acc, off_v, src_v, trg_v,
              zbuf, cbuf, sem1, sem2):
    cid = lax.axis_index("c")
    sid = lax.axis_index("s")
    wid = cid * 16 + sid
    nbase = wid * NR
    lanes = lax.iota(jnp.int32, 16)

    pltpu.sync_copy(ccur.at[pl.ds(nbase, NR)], acc.at[pl.ds(0, NR)])
    pltpu.sync_copy(off_h, off_v)

    def vscalar(idx):
        win = off_v[pl.ds((idx // 16) * 16, 16)]
        sel = jnp.where(lanes == (idx % 16), win, 0)
        return jnp.sum(sel)

    o0 = vscalar(wid)
    o1 = vscalar(wid + 1)
    nchunks = (o1 - o0) // E

    def chunk_body(i, carry):
        cb = pl.multiple_of(o0 + i * E, 8)
        pltpu.sync_copy(src.at[pl.ds(cb, E)], src_v)
        pltpu.sync_copy(trg.at[pl.ds(cb, E)], trg_v)
        cpz = pltpu.async_copy(c0.at[src_v], zbuf, sem1)
        cpc = pltpu.async_copy(ccur.at[trg_v], cbuf, sem2)
        cpz.wait()
        cpc.wait()

        def group_body(g, gc):
            rows = g * 16 + lanes
            tv = trg_v[pl.ds(g * 16, 16)]
            soff = jnp.minimum(jnp.maximum(tv - nbase, 0), NR)

            ps = []
            for f in range(KF):
                def dot_block(b, pacc, f=f):
                    colv = jnp.full((16,), f * DD, jnp.int32) + b * 16
                    acc16 = pacc
                    for j in range(16):
                        col = colv + ((lanes + j) & 15)
                        zc = plsc.load_gather(zbuf, [rows, col])
                        cc = plsc.load_gather(cbuf, [rows, col])
                        acc16 = acc16 + zc * cc
                    return acc16
                ps.append(lax.fori_loop(0, DD // 16, dot_block,
                                        jnp.zeros((16,), jnp.float32)))

            mx = jnp.maximum(jnp.maximum(ps[0], ps[1]),
                             jnp.maximum(ps[2], ps[3]))
            es = [jnp.exp(p - mx) for p in ps]
            ssum = es[0] + es[1] + es[2] + es[3]
            rinv = 1.0 / ssum
            ws = [e * rinv for e in es]

            for f in range(KF):
                def w_block(b, carry2, f=f):
                    colv = jnp.full((16,), f * DD, jnp.int32) + b * 16
                    for j in range(16):
                        col = colv + ((lanes + j) & 15)
                        zc = plsc.load_gather(zbuf, [rows, col])
                        plsc.addupdate_scatter(acc, [soff, col],
                                               zc * ws[f])
                    return carry2
                lax.fori_loop(0, DD // 16, w_block, 0)
            return gc

        lax.fori_loop(0, GROUPS, group_body, 0)
        return carry

    lax.fori_loop(0, nchunks, chunk_body, 0)

    pltpu.sync_copy(acc.at[pl.ds(0, NR)], out.at[pl.ds(nbase, NR)])


def _prepare_edges(trg, src):
    bucket = trg // NR
    key = bucket * 262144 + jnp.arange(M, dtype=jnp.int32)
    ks = jnp.sort(key)
    order = ks & 262143
    b_s = ks >> 18
    start = jnp.searchsorted(b_s, jnp.arange(NT + 1)).astype(jnp.int32)
    counts = start[1:] - start[:-1]
    padded = ((counts + E - 1) // E) * E
    off = jnp.concatenate(
        [jnp.zeros((1,), jnp.int32),
         jnp.cumsum(padded).astype(jnp.int32)])
    j = jnp.arange(M_PAD, dtype=jnp.int32)
    bj = (jnp.searchsorted(off, j, side="right") - 1).astype(jnp.int32)
    gidx = start[bj] + (j - off[bj])
    real = gidx < start[bj + 1]
    gclamp = jnp.minimum(gidx, M - 1)
    trg_sorted = trg[order]
    src_sorted = src[order]
    trg_p = jnp.where(real, trg_sorted[gclamp], DUMMY)
    src_p = jnp.where(real, src_sorted[gclamp], DUMMY)
    off48 = jnp.pad(off, (0, NOFF - NT - 1), mode="edge")
    return trg_p, src_p, off48.astype(jnp.int32)


def kernel(x, src_trg):
    trg = src_trg[0]
    src = src_trg[1]
    trg_p, src_p, off48 = _prepare_edges(trg, src)
    x_pad = jnp.pad(x, ((0, N_PAD - N), (0, 0)))
    c0 = _tc_normalize(x_pad)
    c = c0
    for _ in range(ROUTIT):
        acc = _sc_route(c0, c, src_p, trg_p, off48)
        c = _tc_normalize(acc)
    return c[:N]
